# baseline (speedup 1.0000x reference)
import functools

import jax
import jax.numpy as jnp
from jax import lax
from jax.experimental import pallas as pl
from jax.experimental.pallas import tpu as pltpu
from jax.experimental.pallas import tpu_sc as plsc

_NSAMP = 16384
_V = 100000
_NPAD = 100352
_CHUNK = _NPAD // 16
_VECS = _CHUNK // 16
_SAMP_W = _NSAMP // 32
_GRPS = _SAMP_W // 16


def _sampler(freqs_hbm, u_hbm, out_hbm, cdf_v, u_v, o_v, bnd_v, off_v, cdf_sh):
    c_id = lax.axis_index("c")
    s_id = lax.axis_index("s")
    wid = s_id * 2 + c_id
    base = s_id * _CHUNK

    pltpu.sync_copy(freqs_hbm.at[pl.ds(base, _CHUNK)], cdf_v.at[pl.ds(base, _CHUNK)])
    pltpu.sync_copy(u_hbm.at[pl.ds(wid * _SAMP_W, _SAMP_W)], u_v)

    with jax.named_scope("phase_cumsum"):
        def cs_body(i, carry):
            off = base + i * 16
            v = cdf_v[pl.ds(off, 16)]
            cdf_v[pl.ds(off, 16)] = plsc.cumsum(v) + carry
            return carry + jnp.sum(v)

        lax.fori_loop(0, _VECS, cs_body, jnp.float32(0.0))

    with jax.named_scope("phase_exchange"):
        pltpu.sync_copy(cdf_v.at[pl.ds(base, _CHUNK)], cdf_sh.at[pl.ds(base, _CHUNK)])
        plsc.subcore_barrier()
        pltpu.sync_copy(cdf_sh, cdf_v)

    lanes = lax.iota(jnp.int32, 16)
    ends = plsc.load_gather(cdf_v, [lanes * _CHUNK + (_CHUNK - 1)])
    inc = plsc.cumsum(ends)
    bnd_v[...] = inc
    off_v[...] = inc - ends
    total = jnp.sum(ends)

    def grp_body(g, _):
        t = u_v[pl.ds(g * 16, 16)] * total
        c = jnp.zeros((16,), jnp.int32)
        for b in (8, 4, 2, 1):
            val = plsc.load_gather(bnd_v, [c + (b - 1)])
            c = c + jnp.where(val < t, b, 0)
        c = jnp.minimum(c, 15)
        t2 = t - plsc.load_gather(off_v, [c])
        cbase = c * _CHUNK
        lo = jnp.zeros((16,), jnp.int32)
        for b in (4096, 2048, 1024, 512, 256, 128, 64, 32, 16, 8, 4, 2, 1):
            probe = lo + (b - 1)
            val = plsc.load_gather(cdf_v, [cbase + jnp.minimum(probe, _CHUNK - 1)])
            ok = jnp.logical_and(val < t2, probe < _CHUNK)
            lo = lo + jnp.where(ok, b, 0)
        idx = jnp.minimum(cbase + lo, _V - 1)
        o_v[pl.ds(g * 16, 16)] = idx.astype(jnp.float32)
        return 0

    with jax.named_scope("phase_search"):
        lax.fori_loop(0, _GRPS, grp_body, 0)
    pltpu.sync_copy(o_v, out_hbm.at[pl.ds(wid * _SAMP_W, _SAMP_W)])


def kernel(data, freqs):
    del data
    u = jax.random.uniform(jax.random.key(42), (_NSAMP,), dtype=jnp.float32)
    fpad = jnp.pad(freqs, (0, _NPAD - _V))
    mesh = plsc.VectorSubcoreMesh(core_axis_name="c", subcore_axis_name="s")
    run = pl.kernel(
        _sampler,
        mesh=mesh,
        compiler_params=pltpu.CompilerParams(needs_layout_passes=False),
        out_type=jax.ShapeDtypeStruct((_NSAMP,), jnp.float32),
        scratch_types=[
            pltpu.VMEM((_NPAD,), jnp.float32),
            pltpu.VMEM((_SAMP_W,), jnp.float32),
            pltpu.VMEM((_SAMP_W,), jnp.float32),
            pltpu.VMEM((16,), jnp.float32),
            pltpu.VMEM((16,), jnp.float32),
            pltpu.VMEM_SHARED((_NPAD,), jnp.float32),
        ],
    )
    return run(fpad, u)


# speedup vs baseline: 1.0765x; 1.0765x over previous
import functools

import jax
import jax.numpy as jnp
from jax import lax
from jax.experimental import pallas as pl
from jax.experimental.pallas import tpu as pltpu
from jax.experimental.pallas import tpu_sc as plsc

_NSAMP = 16384
_V = 100000
_NPAD = 100352
_CHUNK = _NPAD // 16
_SEG = _CHUNK // 16
_TAIL = _V - 15 * _CHUNK
_SAMP_W = _NSAMP // 32
_GRPS = _SAMP_W // 16
_UNROLL_G = 4
_UNROLL_K = 8


def _lane_shift_prefix(v, lanes):
    for k in (1, 2, 4, 8):
        sh_idx = jnp.maximum(lanes - k, 0)
        dnums = lax.GatherDimensionNumbers(
            offset_dims=(), collapsed_slice_dims=(0,), start_index_map=(0,))
        sh = lax.gather(v, sh_idx[:, None], dnums, slice_sizes=(1,),
                        mode=lax.GatherScatterMode.PROMISE_IN_BOUNDS)
        v = v + jnp.where(lanes >= k, sh, jnp.float32(0.0))
    return v


def _sampler(freqs_hbm, u_hbm, out_hbm, cdf_v, u_v, o_v, bnd_v, off_v, cdf_sh):
    c_id = lax.axis_index("c")
    s_id = lax.axis_index("s")
    wid = s_id * 2 + c_id
    base = s_id * _CHUNK
    lanes = lax.iota(jnp.int32, 16)

    @pl.when(s_id == 15)
    def _():
        pltpu.sync_copy(freqs_hbm.at[pl.ds(base, _TAIL)],
                        cdf_v.at[pl.ds(base, _TAIL)])
        for z in range((_CHUNK - _TAIL) // 16):
            cdf_v[pl.ds(base + _TAIL + z * 16, 16)] = jnp.zeros((16,), jnp.float32)

    @pl.when(s_id != 15)
    def _():
        pltpu.sync_copy(freqs_hbm.at[pl.ds(base, _CHUNK)],
                        cdf_v.at[pl.ds(base, _CHUNK)])

    pltpu.sync_copy(u_hbm.at[pl.ds(wid * _SAMP_W, _SAMP_W)], u_v)

    with jax.named_scope("phase_cumsum"):
        seg_base = base + lanes * _SEG

        def pass_a(i, acc):
            for d in range(_UNROLL_K):
                idxv = seg_base + (i * _UNROLL_K + d)
                acc = acc + plsc.load_gather(cdf_v, [idxv])
                plsc.store_scatter(cdf_v, [idxv], acc)
            return acc

        segtot = lax.fori_loop(0, _SEG // _UNROLL_K, pass_a,
                               jnp.zeros((16,), jnp.float32))
        seg_excl = _lane_shift_prefix(segtot, lanes) - segtot

        def pass_b(i, _):
            for d in range(_UNROLL_K):
                idxv = seg_base + (i * _UNROLL_K + d)
                v = plsc.load_gather(cdf_v, [idxv]) + seg_excl
                plsc.store_scatter(cdf_v, [idxv], v)
            return 0

        lax.fori_loop(0, _SEG // _UNROLL_K, pass_b, 0)

    with jax.named_scope("phase_exchange"):
        pltpu.sync_copy(cdf_v.at[pl.ds(base, _CHUNK)],
                        cdf_sh.at[pl.ds(base, _CHUNK)])
        plsc.subcore_barrier()
        pltpu.sync_copy(cdf_sh, cdf_v)

    ends = plsc.load_gather(cdf_v, [lanes * _CHUNK + (_CHUNK - 1)])
    inc = plsc.cumsum(ends)
    bnd_v[...] = inc
    off_v[...] = inc - ends
    total = jnp.sum(ends)

    def search16(t):
        c = jnp.zeros((16,), jnp.int32)
        for b in (8, 4, 2, 1):
            val = plsc.load_gather(bnd_v, [c + (b - 1)])
            c = c + jnp.where(val < t, b, 0)
        c = jnp.minimum(c, 15)
        t2 = t - plsc.load_gather(off_v, [c])
        cbase = c * _CHUNK
        lo = jnp.zeros((16,), jnp.int32)
        for b in (4096, 2048, 1024, 512, 256, 128, 64, 32, 16, 8, 4, 2, 1):
            probe = jnp.minimum(lo + (b - 1), _CHUNK - 1)
            val = plsc.load_gather(cdf_v, [cbase + probe])
            lo = lo + jnp.where(val < t2, b, 0)
        return jnp.minimum(cbase + lo, _V - 1)

    with jax.named_scope("phase_search"):
        def grp_body(g, _):
            ts = [u_v[pl.ds((g * _UNROLL_G + d) * 16, 16)] * total
                  for d in range(_UNROLL_G)]
            idxs = [search16(t) for t in ts]
            for d in range(_UNROLL_G):
                o_v[pl.ds((g * _UNROLL_G + d) * 16, 16)] = (
                    idxs[d].astype(jnp.float32))
            return 0

        lax.fori_loop(0, _GRPS // _UNROLL_G, grp_body, 0)

    pltpu.sync_copy(o_v, out_hbm.at[pl.ds(wid * _SAMP_W, _SAMP_W)])


def kernel(data, freqs):
    del data
    u = jax.random.uniform(jax.random.key(42), (_NSAMP,), dtype=jnp.float32)
    mesh = plsc.VectorSubcoreMesh(core_axis_name="c", subcore_axis_name="s")
    run = pl.kernel(
        _sampler,
        mesh=mesh,
        compiler_params=pltpu.CompilerParams(needs_layout_passes=False),
        out_type=jax.ShapeDtypeStruct((_NSAMP,), jnp.float32),
        scratch_types=[
            pltpu.VMEM((_NPAD,), jnp.float32),
            pltpu.VMEM((_SAMP_W,), jnp.float32),
            pltpu.VMEM((_SAMP_W,), jnp.float32),
            pltpu.VMEM((16,), jnp.float32),
            pltpu.VMEM((16,), jnp.float32),
            pltpu.VMEM_SHARED((_NPAD,), jnp.float32),
        ],
    )
    return run(freqs, u)
